# baseline (speedup 1.0000x reference)
import functools

import jax
import jax.numpy as jnp
from jax import lax
from jax.experimental import pallas as pl
from jax.experimental.pallas import tpu as pltpu
from jax.experimental.pallas import tpu_sc as plsc

N = 25000
D = 128
E = 200000
NT = 16
NPAD = 25088
EPAD = 204800
ROWS_PT = NPAD // NT
EDG_PT = EPAD // NT
CH = 128
NCHUNK = EDG_PT // CH
TRASH = NPAD - N
BR = 256
NB = NPAD // BR

NLOC = NPAD // 2
TR2 = 256
ACCR = NLOC + TR2
CH2 = 64
NCH2 = EDG_PT // CH2
ECH2 = EPAD // CH2
NBC = 40
ZPT = ACCR // NT
WPT = NLOC // NT

_MESH = plsc.VectorSubcoreMesh(core_axis_name="c", subcore_axis_name="s")

_NZ = ROWS_PT // CH
_ZREM = ROWS_PT - _NZ * CH
_ECH = EPAD // CH
_LAG = 8

_NZ2 = ZPT // CH2
_ZREM2 = ZPT - _NZ2 * CH2


def _deg_body(idx2d, out, onesv, zbuf, ibuf, acc0, acc1, acc2, ssem):
    c = lax.axis_index("c")
    s = lax.axis_index("s")
    accs = [acc0, acc1, acc2]
    one16 = jnp.ones((16,), jnp.float32)
    zero16 = jnp.zeros((16,), jnp.float32)

    def fill(i, _):
        onesv[i] = one16
        zbuf[i] = zero16
        return 0

    lax.fori_loop(0, CH, fill, 0)

    row0 = s * ROWS_PT
    for a in range(3):
        arr = c * 3 + a
        acc = accs[a]
        for k in range(_NZ):
            pltpu.async_copy(zbuf, acc.at[pl.ds(row0 + k * CH, CH)], ssem)
        pltpu.async_copy(zbuf.at[pl.ds(0, _ZREM)],
                         acc.at[pl.ds(row0 + _NZ * CH, _ZREM)], ssem)
        pltpu.sync_copy(idx2d.at[pl.ds(arr * _ECH + s * NCHUNK, NCHUNK)],
                        ibuf)
        for k in range(_NZ):
            pltpu.make_async_copy(zbuf, acc.at[pl.ds(0, CH)], ssem).wait()
        pltpu.make_async_copy(zbuf.at[pl.ds(0, _ZREM)],
                              acc.at[pl.ds(0, _ZREM)], ssem).wait()
        plsc.subcore_barrier()

        def chunk(i, _):
            pltpu.async_copy(onesv, acc.at[ibuf.at[i]], ssem, add=True)

            @pl.when(i >= _LAG)
            def _():
                pltpu.make_async_copy(onesv, acc.at[pl.ds(0, CH)],
                                      ssem).wait()

            return 0

        lax.fori_loop(0, NCHUNK, chunk, 0)
        for _ in range(_LAG):
            pltpu.make_async_copy(onesv, acc.at[pl.ds(0, CH)], ssem).wait()
        plsc.subcore_barrier()
        pltpu.sync_copy(acc.at[pl.ds(row0, ROWS_PT)],
                        out.at[pl.ds(arr * NPAD + row0, ROWS_PT)])


_deg_kernel = pl.kernel(
    _deg_body,
    out_type=jax.ShapeDtypeStruct((6 * NPAD, 16), jnp.float32),
    mesh=_MESH,
    scratch_types=[
        pltpu.VMEM((CH, 16), jnp.float32),
        pltpu.VMEM((CH, 16), jnp.float32),
        pltpu.VMEM((NCHUNK, CH), jnp.int32),
        pltpu.VMEM_SHARED((NPAD, 16), jnp.float32),
        pltpu.VMEM_SHARED((NPAD, 16), jnp.float32),
        pltpu.VMEM_SHARED((NPAD, 16), jnp.float32),
        pltpu.SemaphoreType.DMA,
    ],
    compiler_params=pltpu.CompilerParams(use_tc_tiling_on_sc=False),
)


def _conv_body(table, src2d, dst2d, out, sbuf, dbuf, st0, st1, acc,
               gsem, ssem):
    c = lax.axis_index("c")
    s = lax.axis_index("s")
    zero16 = jnp.zeros((16,), jnp.float32)

    def zfill(i, _):
        for k in range(D // 16):
            st0[i, pl.ds(k * 16, 16)] = zero16
        return 0

    lax.fori_loop(0, CH2, zfill, 0)

    rowz = s * ZPT
    for k in range(_NZ2):
        pltpu.async_copy(st0, acc.at[pl.ds(rowz + k * CH2, CH2)], ssem)
    pltpu.async_copy(st0.at[pl.ds(0, _ZREM2)],
                     acc.at[pl.ds(rowz + _NZ2 * CH2, _ZREM2)], ssem)
    for k in range(_NZ2):
        pltpu.make_async_copy(st0, acc.at[pl.ds(0, CH2)], ssem).wait()
    pltpu.make_async_copy(st0.at[pl.ds(0, _ZREM2)],
                          acc.at[pl.ds(0, _ZREM2)], ssem).wait()
    plsc.subcore_barrier()

    def gat(i, buf):
        pltpu.async_copy(table.at[sbuf.at[i]], buf, gsem)

    def sca(i, buf):
        pltpu.async_copy(buf, acc.at[dbuf.at[i]], ssem, add=True)

    def wait_g():
        pltpu.make_async_copy(table.at[pl.ds(0, CH2)], st0, gsem).wait()

    def wait_s():
        pltpu.make_async_copy(st0, acc.at[pl.ds(0, CH2)], ssem).wait()

    for h in range(NCH2 // NBC):
        pltpu.sync_copy(src2d.at[pl.ds(s * NCH2 + h * NBC, NBC)], sbuf)
        pltpu.sync_copy(
            dst2d.at[pl.ds(c * ECH2 + s * NCH2 + h * NBC, NBC)], dbuf)
        gat(0, st0)

        def pair(k, _):
            i0 = 2 * k

            @pl.when(k > 0)
            def _():
                wait_s()
            gat(i0 + 1, st1)
            wait_g()
            sca(i0, st0)
            wait_s()
            @pl.when(i0 + 2 < NBC)
            def _():
                gat(i0 + 2, st0)
            wait_g()
            sca(i0 + 1, st1)
            return 0

        lax.fori_loop(0, NBC // 2, pair, 0)
        wait_s()
    plsc.subcore_barrier()
    pltpu.sync_copy(acc.at[pl.ds(s * WPT, WPT)],
                    out.at[pl.ds(c * NLOC + s * WPT, WPT)])


_conv_kernel = pl.kernel(
    _conv_body,
    out_type=jax.ShapeDtypeStruct((NPAD, D), jnp.float32),
    mesh=_MESH,
    scratch_types=[
        pltpu.VMEM((NBC, CH2), jnp.int32),
        pltpu.VMEM((NBC, CH2), jnp.int32),
        pltpu.VMEM((CH2, D), jnp.float32),
        pltpu.VMEM((CH2, D), jnp.float32),
        pltpu.VMEM_SHARED((ACCR, D), jnp.float32),
        pltpu.SemaphoreType.DMA,
        pltpu.SemaphoreType.DMA,
    ],
)


_SPEC_X = pl.BlockSpec((BR, D), lambda b: (b, 0))
_SPEC_DEG = pl.BlockSpec((BR, 16), lambda b: (b, 0))
_SPEC_W = pl.BlockSpec((D, D), lambda b: (0, 0))
_SPEC_B = pl.BlockSpec((1, D), lambda b: (0, 0))
_TX = jax.ShapeDtypeStruct((NPAD, D), jnp.float32)


def _norm_blk(deg_ref):
    return lax.rsqrt(jnp.maximum(deg_ref[:, 0:1], 1.0))


def _path(mref, dref, wref, bref):
    return (jnp.dot(mref[...] * _norm_blk(dref), wref[...],
                    preferred_element_type=jnp.float32)
            + bref[...])


def _prep_body(n_out, x_ref, *rest):
    deg_refs = rest[:n_out]
    out_refs = rest[n_out:]
    x = x_ref[...]
    for dref, oref in zip(deg_refs, out_refs):
        oref[...] = x * _norm_blk(dref)


def _prep(x, degs):
    n_out = len(degs)
    outs = pl.pallas_call(
        functools.partial(_prep_body, n_out),
        grid=(NB,),
        in_specs=[_SPEC_X] + [_SPEC_DEG] * n_out,
        out_specs=[_SPEC_X] * n_out,
        out_shape=[_TX] * n_out,
    )(x, *degs)
    return list(outs) if isinstance(outs, (list, tuple)) else [outs]


def _dense_body(nrel, nscale, relu, *refs):
    i = 0
    acc = None
    for _ in range(nrel):
        mref, dref, wref, bref = refs[i:i + 4]
        i += 4
        part = _path(mref, dref, wref, bref)
        acc = part if acc is None else acc + part
    if relu:
        acc = jnp.maximum(acc, 0.0)
    if nscale == 0:
        refs[i][...] = acc
    else:
        sdegs = refs[i:i + nscale]
        outs = refs[i + nscale:]
        for dref, oref in zip(sdegs, outs):
            oref[...] = acc * _norm_blk(dref)


def _dense(ms, degs_in, Ws, bs, scale_degs):
    nrel = len(ms)
    final = scale_degs is None
    nscale = 0 if final else len(scale_degs)
    in_specs = []
    args = []
    for m, dgr, w, b in zip(ms, degs_in, Ws, bs):
        in_specs += [_SPEC_X, _SPEC_DEG, _SPEC_W, _SPEC_B]
        args += [m, dgr, w, b.reshape(1, D)]
    if final:
        out_specs = _SPEC_X
        out_shape = _TX
    else:
        in_specs += [_SPEC_DEG] * nscale
        args += list(scale_degs)
        out_specs = [_SPEC_X] * nscale
        out_shape = [_TX] * nscale
    res = pl.pallas_call(
        functools.partial(_dense_body, nrel, nscale, not final),
        grid=(NB,),
        in_specs=in_specs,
        out_specs=out_specs,
        out_shape=out_shape,
    )(*args)
    if final:
        return res
    return list(res) if isinstance(res, (list, tuple)) else [res]


def kernel(x_user, x_item, ei_clicks, ei_clicked_by, ei_follows,
           W1_clicks, b1_clicks, W1_clicked_by, b1_clicked_by,
           W1_follows, b1_follows,
           W2_clicks, b2_clicks, W2_clicked_by, b2_clicked_by,
           W2_follows, b2_follows):
    pad_idx = N + (jnp.arange(EPAD - E, dtype=jnp.int32) % TRASH)

    def pad_e(a):
        return jnp.concatenate([a.astype(jnp.int32), pad_idx])

    s_cl, d_cl = pad_e(ei_clicks[0]), pad_e(ei_clicks[1])
    s_cb, d_cb = pad_e(ei_clicked_by[0]), pad_e(ei_clicked_by[1])
    s_fl, d_fl = pad_e(ei_follows[0]), pad_e(ei_follows[1])

    idx_all = jnp.concatenate([s_cl, d_cl, s_cb, d_cb, s_fl, d_fl])
    degflat = _deg_kernel(idx_all.reshape(6 * _ECH, CH))

    def dg(a):
        return degflat[a * NPAD:(a + 1) * NPAD]

    spread = NLOC + (jnp.arange(EPAD, dtype=jnp.int32) % TR2)

    def src2d(a):
        return a.reshape(ECH2, CH2)

    def dstloc(a):
        halves = []
        for core in (0, 1):
            v = a - core * NLOC
            ok = (v >= 0) & (v < NLOC)
            halves.append(jnp.where(ok, v, spread))
        return jnp.concatenate(halves).reshape(2 * ECH2, CH2)

    s_cl, s_cb, s_fl = src2d(s_cl), src2d(s_cb), src2d(s_fl)
    d_cl, d_cb, d_fl = dstloc(d_cl), dstloc(d_cb), dstloc(d_fl)

    t1_cl, t1_fl = _prep(x_user, [dg(0), dg(4)])
    (t1_cb,) = _prep(x_item, [dg(2)])

    m1_cl = _conv_kernel(t1_cl, s_cl, d_cl)
    m1_cb = _conv_kernel(t1_cb, s_cb, d_cb)
    m1_fl = _conv_kernel(t1_fl, s_fl, d_fl)

    (t2_cb,) = _dense([m1_cl], [dg(1)], [W1_clicks], [b1_clicks], [dg(2)])
    t2_cl, t2_fl = _dense([m1_cb, m1_fl], [dg(3), dg(5)],
                          [W1_clicked_by, W1_follows],
                          [b1_clicked_by, b1_follows], [dg(0), dg(4)])

    m2_cl = _conv_kernel(t2_cl, s_cl, d_cl)
    m2_cb = _conv_kernel(t2_cb, s_cb, d_cb)
    m2_fl = _conv_kernel(t2_fl, s_fl, d_fl)

    h_item2 = _dense([m2_cl], [dg(1)], [W2_clicks], [b2_clicks], None)[:N]
    h_user2 = _dense([m2_cb, m2_fl], [dg(3), dg(5)],
                     [W2_clicked_by, W2_follows],
                     [b2_clicked_by, b2_follows], None)[:N]
    return (h_user2, h_item2)


# speedup vs baseline: 1.3025x; 1.3025x over previous
import functools

import jax
import jax.numpy as jnp
from jax import lax
from jax.experimental import pallas as pl
from jax.experimental.pallas import tpu as pltpu
from jax.experimental.pallas import tpu_sc as plsc

N = 25000
D = 128
DH = 64
E = 200000
NT = 16
NPAD = 25088
EPAD = 204800
ROWS_PT = NPAD // NT
EDG_PT = EPAD // NT
CH = 128
NCHUNK = EDG_PT // CH
ZR = 224
TRASH = NPAD - N
BR = 256
NB = NPAD // BR

_MESH = plsc.VectorSubcoreMesh(core_axis_name="c", subcore_axis_name="s")


_NZ = ROWS_PT // CH
_ZREM = ROWS_PT - _NZ * CH
_ECH = EPAD // CH
_LAG = 8
_NH = NCHUNK // 2


def _deg_body(idx2d, out, onesv, zbuf, ibuf, acc0, acc1, acc2, ssem):
    c = lax.axis_index("c")
    s = lax.axis_index("s")
    accs = [acc0, acc1, acc2]
    one16 = jnp.ones((16,), jnp.float32)
    zero16 = jnp.zeros((16,), jnp.float32)

    def fill(i, _):
        onesv[i] = one16
        zbuf[i] = zero16
        return 0

    lax.fori_loop(0, CH, fill, 0)

    row0 = s * ROWS_PT
    for a in range(3):
        arr = c * 3 + a
        acc = accs[a]
        for k in range(_NZ):
            pltpu.async_copy(zbuf, acc.at[pl.ds(row0 + k * CH, CH)], ssem)
        pltpu.async_copy(zbuf.at[pl.ds(0, _ZREM)],
                         acc.at[pl.ds(row0 + _NZ * CH, _ZREM)], ssem)
        pltpu.sync_copy(idx2d.at[pl.ds(arr * _ECH + s * NCHUNK, NCHUNK)],
                        ibuf)
        for k in range(_NZ):
            pltpu.make_async_copy(zbuf, acc.at[pl.ds(0, CH)], ssem).wait()
        pltpu.make_async_copy(zbuf.at[pl.ds(0, _ZREM)],
                              acc.at[pl.ds(0, _ZREM)], ssem).wait()
        plsc.subcore_barrier()

        def chunk(i, _):
            pltpu.async_copy(onesv, acc.at[ibuf.at[i]], ssem, add=True)

            @pl.when(i >= _LAG)
            def _():
                pltpu.make_async_copy(onesv, acc.at[pl.ds(0, CH)],
                                      ssem).wait()

            return 0

        lax.fori_loop(0, NCHUNK, chunk, 0)
        for _ in range(_LAG):
            pltpu.make_async_copy(onesv, acc.at[pl.ds(0, CH)], ssem).wait()
        plsc.subcore_barrier()
        pltpu.sync_copy(acc.at[pl.ds(row0, ROWS_PT)],
                        out.at[pl.ds(arr * NPAD + row0, ROWS_PT)])


_deg_kernel = pl.kernel(
    _deg_body,
    out_type=jax.ShapeDtypeStruct((6 * NPAD, 16), jnp.float32),
    mesh=_MESH,
    scratch_types=[
        pltpu.VMEM((CH, 16), jnp.float32),
        pltpu.VMEM((CH, 16), jnp.float32),
        pltpu.VMEM((NCHUNK, CH), jnp.int32),
        pltpu.VMEM_SHARED((NPAD, 16), jnp.float32),
        pltpu.VMEM_SHARED((NPAD, 16), jnp.float32),
        pltpu.VMEM_SHARED((NPAD, 16), jnp.float32),
        pltpu.SemaphoreType.DMA,
    ],
    compiler_params=pltpu.CompilerParams(use_tc_tiling_on_sc=False),
)


def _conv_body(table, so2d, dst2d, out, sbuf, dbuf, st0, st1, acc,
               gsem, ssem):
    c = lax.axis_index("c")
    s = lax.axis_index("s")
    zero16 = jnp.zeros((16,), jnp.float32)
    row0 = s * ROWS_PT

    def wait_g():
        pltpu.make_async_copy(table.at[pl.ds(0, CH)], st0, gsem).wait()

    def wait_s():
        pltpu.make_async_copy(st0, acc.at[pl.ds(0, CH)], ssem).wait()

    if True:
        def zfill(i, _):
            for k in range(DH // 16):
                st0[i, pl.ds(k * 16, 16)] = zero16
            return 0

        lax.fori_loop(0, CH, zfill, 0)

        for k in range(_NZ):
            pltpu.async_copy(st0, acc.at[pl.ds(row0 + k * CH, CH)], ssem)
        pltpu.async_copy(st0.at[pl.ds(0, _ZREM)],
                         acc.at[pl.ds(row0 + _NZ * CH, _ZREM)], ssem)
        for k in range(_NZ):
            pltpu.make_async_copy(st0, acc.at[pl.ds(0, CH)], ssem).wait()
        pltpu.make_async_copy(st0.at[pl.ds(0, _ZREM)],
                              acc.at[pl.ds(0, _ZREM)], ssem).wait()
        plsc.subcore_barrier()

        def gat(i, buf):
            pltpu.async_copy(table.at[sbuf.at[i]], buf, gsem)

        def sca(i, buf):
            pltpu.async_copy(buf, acc.at[dbuf.at[i]], ssem, add=True)

        for h in range(2):
            pltpu.sync_copy(
                so2d.at[pl.ds((c * NT + s) * NCHUNK + h * _NH, _NH)], sbuf)
            pltpu.sync_copy(dst2d.at[pl.ds(s * NCHUNK + h * _NH, _NH)],
                            dbuf)
            gat(0, st0)

            def pair(k, _):
                i0 = 2 * k

                @pl.when(k > 0)
                def _():
                    wait_s()
                gat(i0 + 1, st1)
                wait_g()
                sca(i0, st0)
                wait_s()
                @pl.when(i0 + 2 < _NH)
                def _():
                    gat(i0 + 2, st0)
                wait_g()
                sca(i0 + 1, st1)
                return 0

            lax.fori_loop(0, _NH // 2, pair, 0)
            wait_s()
        plsc.subcore_barrier()
        pltpu.sync_copy(acc.at[pl.ds(row0, ROWS_PT)],
                        out.at[pl.ds(c * NPAD + row0, ROWS_PT)])


_conv_kernel = pl.kernel(
    _conv_body,
    out_type=jax.ShapeDtypeStruct((2 * NPAD, DH), jnp.float32),
    mesh=_MESH,
    scratch_types=[
        pltpu.VMEM((_NH, CH), jnp.int32),
        pltpu.VMEM((_NH, CH), jnp.int32),
        pltpu.VMEM((CH, DH), jnp.float32),
        pltpu.VMEM((CH, DH), jnp.float32),
        pltpu.VMEM_SHARED((NPAD, DH), jnp.float32),
        pltpu.SemaphoreType.DMA,
        pltpu.SemaphoreType.DMA,
    ],
    compiler_params=pltpu.CompilerParams(use_tc_tiling_on_sc=False),
)


def _norm_blk(deg_ref):
    return lax.rsqrt(jnp.maximum(deg_ref[:, 0:1], 1.0))


def _split_store(oref, val):
    oref[0] = val[:, 0:DH]
    oref[1] = val[:, DH:D]


_SPEC_M = pl.BlockSpec((2, BR, DH), lambda b: (0, b, 0))
_SPEC_DEG = pl.BlockSpec((BR, 16), lambda b: (b, 0))
_SPEC_W = pl.BlockSpec((D, D), lambda b: (0, 0))
_SPEC_B = pl.BlockSpec((1, D), lambda b: (0, 0))
_SPEC_X = pl.BlockSpec((BR, D), lambda b: (b, 0))
_SPEC_OUT2 = pl.BlockSpec((2, BR, DH), lambda b: (0, b, 0))
_T2 = jax.ShapeDtypeStruct((2, NPAD, DH), jnp.float32)


def _path(mref, dref, wref, bref):
    norm = _norm_blk(dref)
    return (jnp.dot(mref[0] * norm, wref[0:DH, :],
                    preferred_element_type=jnp.float32)
            + jnp.dot(mref[1] * norm, wref[DH:D, :],
                      preferred_element_type=jnp.float32)
            + bref[...])


def _prep_body(n_out, x_ref, *rest):
    deg_refs = rest[:n_out]
    out_refs = rest[n_out:]
    x = x_ref[...]
    for dref, oref in zip(deg_refs, out_refs):
        _split_store(oref, x * _norm_blk(dref))


def _prep(x, degs):
    n_out = len(degs)
    outs = pl.pallas_call(
        functools.partial(_prep_body, n_out),
        grid=(NB,),
        in_specs=[_SPEC_X] + [_SPEC_DEG] * n_out,
        out_specs=[_SPEC_OUT2] * n_out,
        out_shape=[_T2] * n_out,
    )(x, *degs)
    outs = outs if isinstance(outs, (list, tuple)) else [outs]
    return [o.reshape(2 * NPAD, DH) for o in outs]


def _dense_body(nrel, nscale, relu, *refs):
    i = 0
    acc = None
    for _ in range(nrel):
        mref, dref, wref, bref = refs[i:i + 4]
        i += 4
        part = _path(mref, dref, wref, bref)
        acc = part if acc is None else acc + part
    if relu:
        acc = jnp.maximum(acc, 0.0)
    if nscale == 0:
        refs[i][...] = acc
    else:
        sdegs = refs[i:i + nscale]
        outs = refs[i + nscale:]
        for dref, oref in zip(sdegs, outs):
            _split_store(oref, acc * _norm_blk(dref))


def _dense(ms, degs_in, Ws, bs, scale_degs):
    nrel = len(ms)
    final = scale_degs is None
    nscale = 0 if final else len(scale_degs)
    in_specs = []
    args = []
    for m, dgr, w, b in zip(ms, degs_in, Ws, bs):
        in_specs += [_SPEC_M, _SPEC_DEG, _SPEC_W, _SPEC_B]
        args += [m.reshape(2, NPAD, DH), dgr, w, b.reshape(1, D)]
    if final:
        out_specs = pl.BlockSpec((BR, D), lambda b: (b, 0))
        out_shape = jax.ShapeDtypeStruct((NPAD, D), jnp.float32)
    else:
        in_specs += [_SPEC_DEG] * nscale
        args += list(scale_degs)
        out_specs = [_SPEC_OUT2] * nscale
        out_shape = [_T2] * nscale
    res = pl.pallas_call(
        functools.partial(_dense_body, nrel, nscale, not final),
        grid=(NB,),
        in_specs=in_specs,
        out_specs=out_specs,
        out_shape=out_shape,
    )(*args)
    if final:
        return res
    res = res if isinstance(res, (list, tuple)) else [res]
    return [o.reshape(2 * NPAD, DH) for o in res]


def kernel(x_user, x_item, ei_clicks, ei_clicked_by, ei_follows,
           W1_clicks, b1_clicks, W1_clicked_by, b1_clicked_by,
           W1_follows, b1_follows,
           W2_clicks, b2_clicks, W2_clicked_by, b2_clicked_by,
           W2_follows, b2_follows):
    pad_idx = N + (jnp.arange(EPAD - E, dtype=jnp.int32) % TRASH)

    def pad_e(a):
        return jnp.concatenate([a.astype(jnp.int32), pad_idx])

    s_cl, d_cl = pad_e(ei_clicks[0]), pad_e(ei_clicks[1])
    s_cb, d_cb = pad_e(ei_clicked_by[0]), pad_e(ei_clicked_by[1])
    s_fl, d_fl = pad_e(ei_follows[0]), pad_e(ei_follows[1])

    idx_all = jnp.concatenate([s_cl, d_cl, s_cb, d_cb, s_fl, d_fl])
    degflat = _deg_kernel(idx_all.reshape(6 * _ECH, CH))

    def src2d(a):
        return jnp.concatenate([a, a + NPAD]).reshape(2 * _ECH, CH)

    def dst2d(a):
        return a.reshape(_ECH, CH)

    s_cl, s_cb, s_fl = src2d(s_cl), src2d(s_cb), src2d(s_fl)
    d_cl, d_cb, d_fl = dst2d(d_cl), dst2d(d_cb), dst2d(d_fl)

    def dg(a):
        return degflat[a * NPAD:(a + 1) * NPAD]

    t1_cl, t1_fl = _prep(x_user, [dg(0), dg(4)])
    (t1_cb,) = _prep(x_item, [dg(2)])

    m1_cl = _conv_kernel(t1_cl, s_cl, d_cl)
    m1_cb = _conv_kernel(t1_cb, s_cb, d_cb)
    m1_fl = _conv_kernel(t1_fl, s_fl, d_fl)

    (t2_cb,) = _dense([m1_cl], [dg(1)], [W1_clicks], [b1_clicks], [dg(2)])
    t2_cl, t2_fl = _dense([m1_cb, m1_fl], [dg(3), dg(5)],
                          [W1_clicked_by, W1_follows],
                          [b1_clicked_by, b1_follows], [dg(0), dg(4)])

    m2_cl = _conv_kernel(t2_cl, s_cl, d_cl)
    m2_cb = _conv_kernel(t2_cb, s_cb, d_cb)
    m2_fl = _conv_kernel(t2_fl, s_fl, d_fl)

    h_item2 = _dense([m2_cl], [dg(1)], [W2_clicks], [b2_clicks], None)[:N]
    h_user2 = _dense([m2_cb, m2_fl], [dg(3), dg(5)],
                     [W2_clicked_by, W2_follows],
                     [b2_clicked_by, b2_follows], None)[:N]
    return (h_user2, h_item2)


# speedup vs baseline: 1.4334x; 1.1005x over previous
import functools

import jax
import jax.numpy as jnp
from jax import lax
from jax.experimental import pallas as pl
from jax.experimental.pallas import tpu as pltpu
from jax.experimental.pallas import tpu_sc as plsc

N = 25000
D = 128
DH = 64
E = 200000
NT = 16
NPAD = 25088
EPAD = 204800
ROWS_PT = NPAD // NT
EDG_PT = EPAD // NT
CH = 128
NCHUNK = EDG_PT // CH
ZR = 224
TRASH = NPAD - N
BR = 512
NB = NPAD // BR

_MESH = plsc.VectorSubcoreMesh(core_axis_name="c", subcore_axis_name="s")


_NZ = ROWS_PT // CH
_ZREM = ROWS_PT - _NZ * CH
_ECH = EPAD // CH
_LAG = 8
_NH = NCHUNK // 2


def _deg_body(idx2d, out, onesv, zbuf, ibuf, acc0, acc1, acc2, ssem):
    c = lax.axis_index("c")
    s = lax.axis_index("s")
    accs = [acc0, acc1, acc2]
    one16 = jnp.ones((16,), jnp.float32)
    zero16 = jnp.zeros((16,), jnp.float32)

    def fill(i, _):
        onesv[i] = one16
        zbuf[i] = zero16
        return 0

    lax.fori_loop(0, CH, fill, 0)

    row0 = s * ROWS_PT
    for a in range(3):
        arr = c * 3 + a
        acc = accs[a]
        for k in range(_NZ):
            pltpu.async_copy(zbuf, acc.at[pl.ds(row0 + k * CH, CH)], ssem)
        pltpu.async_copy(zbuf.at[pl.ds(0, _ZREM)],
                         acc.at[pl.ds(row0 + _NZ * CH, _ZREM)], ssem)
        pltpu.sync_copy(idx2d.at[pl.ds(arr * _ECH + s * NCHUNK, NCHUNK)],
                        ibuf)
        for k in range(_NZ):
            pltpu.make_async_copy(zbuf, acc.at[pl.ds(0, CH)], ssem).wait()
        pltpu.make_async_copy(zbuf.at[pl.ds(0, _ZREM)],
                              acc.at[pl.ds(0, _ZREM)], ssem).wait()
        plsc.subcore_barrier()

        def chunk(i, _):
            pltpu.async_copy(onesv, acc.at[ibuf.at[i]], ssem, add=True)

            @pl.when(i >= _LAG)
            def _():
                pltpu.make_async_copy(onesv, acc.at[pl.ds(0, CH)],
                                      ssem).wait()

            return 0

        lax.fori_loop(0, NCHUNK, chunk, 0)
        for _ in range(_LAG):
            pltpu.make_async_copy(onesv, acc.at[pl.ds(0, CH)], ssem).wait()
        plsc.subcore_barrier()
        pltpu.sync_copy(acc.at[pl.ds(row0, ROWS_PT)],
                        out.at[pl.ds(arr * NPAD + row0, ROWS_PT)])


_deg_kernel = pl.kernel(
    _deg_body,
    out_type=jax.ShapeDtypeStruct((6 * NPAD, 16), jnp.float32),
    mesh=_MESH,
    scratch_types=[
        pltpu.VMEM((CH, 16), jnp.float32),
        pltpu.VMEM((CH, 16), jnp.float32),
        pltpu.VMEM((NCHUNK, CH), jnp.int32),
        pltpu.VMEM_SHARED((NPAD, 16), jnp.float32),
        pltpu.VMEM_SHARED((NPAD, 16), jnp.float32),
        pltpu.VMEM_SHARED((NPAD, 16), jnp.float32),
        pltpu.SemaphoreType.DMA,
    ],
    compiler_params=pltpu.CompilerParams(use_tc_tiling_on_sc=False),
)


def _conv_body(table, so2d, dst2d, out, sbuf, dbuf, st0, st1, acc,
               gsem, ssem):
    c = lax.axis_index("c")
    s = lax.axis_index("s")
    zero16 = jnp.zeros((16,), jnp.float32)
    row0 = s * ROWS_PT

    def wait_g():
        pltpu.make_async_copy(table.at[pl.ds(0, CH)], st0, gsem).wait()

    def wait_s():
        pltpu.make_async_copy(st0, acc.at[pl.ds(0, CH)], ssem).wait()

    if True:
        def zfill(i, _):
            for k in range(DH // 16):
                st0[i, pl.ds(k * 16, 16)] = zero16
            return 0

        lax.fori_loop(0, CH, zfill, 0)

        for k in range(_NZ):
            pltpu.async_copy(st0, acc.at[pl.ds(row0 + k * CH, CH)], ssem)
        pltpu.async_copy(st0.at[pl.ds(0, _ZREM)],
                         acc.at[pl.ds(row0 + _NZ * CH, _ZREM)], ssem)
        for k in range(_NZ):
            pltpu.make_async_copy(st0, acc.at[pl.ds(0, CH)], ssem).wait()
        pltpu.make_async_copy(st0.at[pl.ds(0, _ZREM)],
                              acc.at[pl.ds(0, _ZREM)], ssem).wait()
        plsc.subcore_barrier()

        def gat(i, buf):
            pltpu.async_copy(table.at[sbuf.at[i]], buf, gsem)

        def sca(i, buf):
            pltpu.async_copy(buf, acc.at[dbuf.at[i]], ssem, add=True)

        for h in range(2):
            pltpu.sync_copy(
                so2d.at[pl.ds((c * NT + s) * NCHUNK + h * _NH, _NH)], sbuf)
            pltpu.sync_copy(dst2d.at[pl.ds(s * NCHUNK + h * _NH, _NH)],
                            dbuf)
            gat(0, st0)

            def pair(k, _):
                i0 = 2 * k

                @pl.when(k > 0)
                def _():
                    wait_s()
                gat(i0 + 1, st1)
                wait_g()
                sca(i0, st0)
                wait_s()
                @pl.when(i0 + 2 < _NH)
                def _():
                    gat(i0 + 2, st0)
                wait_g()
                sca(i0 + 1, st1)
                return 0

            lax.fori_loop(0, _NH // 2, pair, 0)
            wait_s()
        plsc.subcore_barrier()
        pltpu.sync_copy(acc.at[pl.ds(row0, ROWS_PT)],
                        out.at[pl.ds(c * NPAD + row0, ROWS_PT)])


_conv_kernel = pl.kernel(
    _conv_body,
    out_type=jax.ShapeDtypeStruct((2 * NPAD, DH), jnp.float32),
    mesh=_MESH,
    scratch_types=[
        pltpu.VMEM((_NH, CH), jnp.int32),
        pltpu.VMEM((_NH, CH), jnp.int32),
        pltpu.VMEM((CH, DH), jnp.float32),
        pltpu.VMEM((CH, DH), jnp.float32),
        pltpu.VMEM_SHARED((NPAD, DH), jnp.float32),
        pltpu.SemaphoreType.DMA,
        pltpu.SemaphoreType.DMA,
    ],
    compiler_params=pltpu.CompilerParams(use_tc_tiling_on_sc=False),
)


def _norm_blk(deg_ref):
    return lax.rsqrt(jnp.maximum(deg_ref[:, 0:1], 1.0))


def _split_store(oref, val):
    oref[0] = val[:, 0:DH]
    oref[1] = val[:, DH:D]


_SPEC_M = pl.BlockSpec((2, BR, DH), lambda b: (0, b, 0))
_SPEC_DEG = pl.BlockSpec((BR, 16), lambda b: (b, 0))
_SPEC_W = pl.BlockSpec((D, D), lambda b: (0, 0))
_SPEC_B = pl.BlockSpec((1, D), lambda b: (0, 0))
_SPEC_X = pl.BlockSpec((BR, D), lambda b: (b, 0))
_SPEC_OUT2 = pl.BlockSpec((2, BR, DH), lambda b: (0, b, 0))
_T2 = jax.ShapeDtypeStruct((2, NPAD, DH), jnp.float32)


def _path(mref, dref, wref, bref):
    norm = _norm_blk(dref)
    return (jnp.dot(mref[0] * norm, wref[0:DH, :],
                    preferred_element_type=jnp.float32)
            + jnp.dot(mref[1] * norm, wref[DH:D, :],
                      preferred_element_type=jnp.float32)
            + bref[...])


def _prep_body(n_out, x_ref, *rest):
    deg_refs = rest[:n_out]
    out_refs = rest[n_out:]
    x = x_ref[...]
    for dref, oref in zip(deg_refs, out_refs):
        _split_store(oref, x * _norm_blk(dref))


def _prep(x, degs):
    n_out = len(degs)
    outs = pl.pallas_call(
        functools.partial(_prep_body, n_out),
        grid=(NB,),
        in_specs=[_SPEC_X] + [_SPEC_DEG] * n_out,
        out_specs=[_SPEC_OUT2] * n_out,
        out_shape=[_T2] * n_out,
    )(x, *degs)
    outs = outs if isinstance(outs, (list, tuple)) else [outs]
    return [o.reshape(2 * NPAD, DH) for o in outs]


def _dense_body(nrel, nscale, relu, *refs):
    i = 0
    acc = None
    for _ in range(nrel):
        mref, dref, wref, bref = refs[i:i + 4]
        i += 4
        part = _path(mref, dref, wref, bref)
        acc = part if acc is None else acc + part
    if relu:
        acc = jnp.maximum(acc, 0.0)
    if nscale == 0:
        refs[i][...] = acc
    else:
        sdegs = refs[i:i + nscale]
        outs = refs[i + nscale:]
        for dref, oref in zip(sdegs, outs):
            _split_store(oref, acc * _norm_blk(dref))


def _dense(ms, degs_in, Ws, bs, scale_degs):
    nrel = len(ms)
    final = scale_degs is None
    nscale = 0 if final else len(scale_degs)
    in_specs = []
    args = []
    for m, dgr, w, b in zip(ms, degs_in, Ws, bs):
        in_specs += [_SPEC_M, _SPEC_DEG, _SPEC_W, _SPEC_B]
        args += [m.reshape(2, NPAD, DH), dgr, w, b.reshape(1, D)]
    if final:
        out_specs = pl.BlockSpec((BR, D), lambda b: (b, 0))
        out_shape = jax.ShapeDtypeStruct((NPAD, D), jnp.float32)
    else:
        in_specs += [_SPEC_DEG] * nscale
        args += list(scale_degs)
        out_specs = [_SPEC_OUT2] * nscale
        out_shape = [_T2] * nscale
    res = pl.pallas_call(
        functools.partial(_dense_body, nrel, nscale, not final),
        grid=(NB,),
        in_specs=in_specs,
        out_specs=out_specs,
        out_shape=out_shape,
    )(*args)
    if final:
        return res
    res = res if isinstance(res, (list, tuple)) else [res]
    return [o.reshape(2 * NPAD, DH) for o in res]


def kernel(x_user, x_item, ei_clicks, ei_clicked_by, ei_follows,
           W1_clicks, b1_clicks, W1_clicked_by, b1_clicked_by,
           W1_follows, b1_follows,
           W2_clicks, b2_clicks, W2_clicked_by, b2_clicked_by,
           W2_follows, b2_follows):
    pad_idx = N + (jnp.arange(EPAD - E, dtype=jnp.int32) % TRASH)

    def pad_e(a):
        return jnp.concatenate([a.astype(jnp.int32), pad_idx])

    s_cl, d_cl = pad_e(ei_clicks[0]), pad_e(ei_clicks[1])
    s_cb, d_cb = pad_e(ei_clicked_by[0]), pad_e(ei_clicked_by[1])
    s_fl, d_fl = pad_e(ei_follows[0]), pad_e(ei_follows[1])

    idx_all = jnp.concatenate([s_cl, d_cl, s_cb, d_cb, s_fl, d_fl])
    degflat = _deg_kernel(idx_all.reshape(6 * _ECH, CH))

    def src2d(a):
        return jnp.concatenate([a, a + NPAD]).reshape(2 * _ECH, CH)

    def dst2d(a):
        return a.reshape(_ECH, CH)

    s_cl, s_cb, s_fl = src2d(s_cl), src2d(s_cb), src2d(s_fl)
    d_cl, d_cb, d_fl = dst2d(d_cl), dst2d(d_cb), dst2d(d_fl)

    def dg(a):
        return degflat[a * NPAD:(a + 1) * NPAD]

    t1_cl, t1_fl = _prep(x_user, [dg(0), dg(4)])
    (t1_cb,) = _prep(x_item, [dg(2)])

    m1_cl = _conv_kernel(t1_cl, s_cl, d_cl)
    m1_cb = _conv_kernel(t1_cb, s_cb, d_cb)
    m1_fl = _conv_kernel(t1_fl, s_fl, d_fl)

    (t2_cb,) = _dense([m1_cl], [dg(1)], [W1_clicks], [b1_clicks], [dg(2)])
    t2_cl, t2_fl = _dense([m1_cb, m1_fl], [dg(3), dg(5)],
                          [W1_clicked_by, W1_follows],
                          [b1_clicked_by, b1_follows], [dg(0), dg(4)])

    m2_cl = _conv_kernel(t2_cl, s_cl, d_cl)
    m2_cb = _conv_kernel(t2_cb, s_cb, d_cb)
    m2_fl = _conv_kernel(t2_fl, s_fl, d_fl)

    h_item2 = _dense([m2_cl], [dg(1)], [W2_clicks], [b2_clicks], None)[:N]
    h_user2 = _dense([m2_cb, m2_fl], [dg(3), dg(5)],
                     [W2_clicked_by, W2_follows],
                     [b2_clicked_by, b2_follows], None)[:N]
    return (h_user2, h_item2)


# speedup vs baseline: 1.4565x; 1.0161x over previous
import functools

import jax
import jax.numpy as jnp
from jax import lax
from jax.experimental import pallas as pl
from jax.experimental.pallas import tpu as pltpu
from jax.experimental.pallas import tpu_sc as plsc

N = 25000
D = 128
DH = 64
E = 200000
NT = 16
NPAD = 25088
EPAD = 204800
ROWS_PT = NPAD // NT
EDG_PT = EPAD // NT
CH = 128
NCHUNK = EDG_PT // CH
ZR = 224
TRASH = NPAD - N
BR = 896
NB = NPAD // BR

_MESH = plsc.VectorSubcoreMesh(core_axis_name="c", subcore_axis_name="s")


_NZ = ROWS_PT // CH
_ZREM = ROWS_PT - _NZ * CH
_ECH = EPAD // CH
_LAG = 8
_NH = NCHUNK // 2


def _deg_body(idx2d, out, onesv, zbuf, ibuf, acc0, acc1, acc2, ssem):
    c = lax.axis_index("c")
    s = lax.axis_index("s")
    accs = [acc0, acc1, acc2]
    one16 = jnp.ones((16,), jnp.float32)
    zero16 = jnp.zeros((16,), jnp.float32)

    def fill(i, _):
        onesv[i] = one16
        zbuf[i] = zero16
        return 0

    lax.fori_loop(0, CH, fill, 0)

    row0 = s * ROWS_PT
    for a in range(3):
        arr = c * 3 + a
        acc = accs[a]
        for k in range(_NZ):
            pltpu.async_copy(zbuf, acc.at[pl.ds(row0 + k * CH, CH)], ssem)
        pltpu.async_copy(zbuf.at[pl.ds(0, _ZREM)],
                         acc.at[pl.ds(row0 + _NZ * CH, _ZREM)], ssem)
        pltpu.sync_copy(idx2d.at[pl.ds(arr * _ECH + s * NCHUNK, NCHUNK)],
                        ibuf)
        for k in range(_NZ):
            pltpu.make_async_copy(zbuf, acc.at[pl.ds(0, CH)], ssem).wait()
        pltpu.make_async_copy(zbuf.at[pl.ds(0, _ZREM)],
                              acc.at[pl.ds(0, _ZREM)], ssem).wait()
        plsc.subcore_barrier()

        def chunk(i, _):
            pltpu.async_copy(onesv, acc.at[ibuf.at[i]], ssem, add=True)

            @pl.when(i >= _LAG)
            def _():
                pltpu.make_async_copy(onesv, acc.at[pl.ds(0, CH)],
                                      ssem).wait()

            return 0

        lax.fori_loop(0, NCHUNK, chunk, 0)
        for _ in range(_LAG):
            pltpu.make_async_copy(onesv, acc.at[pl.ds(0, CH)], ssem).wait()
        plsc.subcore_barrier()
        pltpu.sync_copy(acc.at[pl.ds(row0, ROWS_PT)],
                        out.at[pl.ds(arr * NPAD + row0, ROWS_PT)])


_deg_kernel = pl.kernel(
    _deg_body,
    out_type=jax.ShapeDtypeStruct((6 * NPAD, 16), jnp.float32),
    mesh=_MESH,
    scratch_types=[
        pltpu.VMEM((CH, 16), jnp.float32),
        pltpu.VMEM((CH, 16), jnp.float32),
        pltpu.VMEM((NCHUNK, CH), jnp.int32),
        pltpu.VMEM_SHARED((NPAD, 16), jnp.float32),
        pltpu.VMEM_SHARED((NPAD, 16), jnp.float32),
        pltpu.VMEM_SHARED((NPAD, 16), jnp.float32),
        pltpu.SemaphoreType.DMA,
    ],
    compiler_params=pltpu.CompilerParams(use_tc_tiling_on_sc=False),
)


def _conv_body(table, so2d, dst2d, out, sbuf, dbuf, st0, st1, acc,
               gsem, ssem):
    c = lax.axis_index("c")
    s = lax.axis_index("s")
    zero16 = jnp.zeros((16,), jnp.float32)
    row0 = s * ROWS_PT

    def wait_g():
        pltpu.make_async_copy(table.at[pl.ds(0, CH)], st0, gsem).wait()

    def wait_s():
        pltpu.make_async_copy(st0, acc.at[pl.ds(0, CH)], ssem).wait()

    if True:
        def zfill(i, _):
            for k in range(DH // 16):
                st0[i, pl.ds(k * 16, 16)] = zero16
            return 0

        lax.fori_loop(0, CH, zfill, 0)

        for k in range(_NZ):
            pltpu.async_copy(st0, acc.at[pl.ds(row0 + k * CH, CH)], ssem)
        pltpu.async_copy(st0.at[pl.ds(0, _ZREM)],
                         acc.at[pl.ds(row0 + _NZ * CH, _ZREM)], ssem)
        for k in range(_NZ):
            pltpu.make_async_copy(st0, acc.at[pl.ds(0, CH)], ssem).wait()
        pltpu.make_async_copy(st0.at[pl.ds(0, _ZREM)],
                              acc.at[pl.ds(0, _ZREM)], ssem).wait()
        plsc.subcore_barrier()

        def gat(i, buf):
            pltpu.async_copy(table.at[sbuf.at[i]], buf, gsem)

        def sca(i, buf):
            pltpu.async_copy(buf, acc.at[dbuf.at[i]], ssem, add=True)

        for h in range(2):
            pltpu.sync_copy(
                so2d.at[pl.ds((c * NT + s) * NCHUNK + h * _NH, _NH)], sbuf)
            pltpu.sync_copy(dst2d.at[pl.ds(s * NCHUNK + h * _NH, _NH)],
                            dbuf)
            gat(0, st0)

            def pair(k, _):
                i0 = 2 * k

                @pl.when(k > 0)
                def _():
                    wait_s()
                gat(i0 + 1, st1)
                wait_g()
                sca(i0, st0)
                wait_s()
                @pl.when(i0 + 2 < _NH)
                def _():
                    gat(i0 + 2, st0)
                wait_g()
                sca(i0 + 1, st1)
                return 0

            lax.fori_loop(0, _NH // 2, pair, 0)
            wait_s()
        plsc.subcore_barrier()
        pltpu.sync_copy(acc.at[pl.ds(row0, ROWS_PT)],
                        out.at[pl.ds(c * NPAD + row0, ROWS_PT)])


_conv_kernel = pl.kernel(
    _conv_body,
    out_type=jax.ShapeDtypeStruct((2 * NPAD, DH), jnp.float32),
    mesh=_MESH,
    scratch_types=[
        pltpu.VMEM((_NH, CH), jnp.int32),
        pltpu.VMEM((_NH, CH), jnp.int32),
        pltpu.VMEM((CH, DH), jnp.float32),
        pltpu.VMEM((CH, DH), jnp.float32),
        pltpu.VMEM_SHARED((NPAD, DH), jnp.float32),
        pltpu.SemaphoreType.DMA,
        pltpu.SemaphoreType.DMA,
    ],
    compiler_params=pltpu.CompilerParams(use_tc_tiling_on_sc=False),
)


def _norm_blk(deg_ref):
    return lax.rsqrt(jnp.maximum(deg_ref[:, 0:1], 1.0))


def _split_store(oref, val):
    oref[0] = val[:, 0:DH]
    oref[1] = val[:, DH:D]


_SPEC_M = pl.BlockSpec((2, BR, DH), lambda b: (0, b, 0))
_SPEC_DEG = pl.BlockSpec((BR, 16), lambda b: (b, 0))
_SPEC_W = pl.BlockSpec((D, D), lambda b: (0, 0))
_SPEC_B = pl.BlockSpec((1, D), lambda b: (0, 0))
_SPEC_X = pl.BlockSpec((BR, D), lambda b: (b, 0))
_SPEC_OUT2 = pl.BlockSpec((2, BR, DH), lambda b: (0, b, 0))
_T2 = jax.ShapeDtypeStruct((2, NPAD, DH), jnp.float32)


def _path(mref, dref, wref, bref):
    norm = _norm_blk(dref)
    return (jnp.dot(mref[0] * norm, wref[0:DH, :],
                    preferred_element_type=jnp.float32)
            + jnp.dot(mref[1] * norm, wref[DH:D, :],
                      preferred_element_type=jnp.float32)
            + bref[...])


def _prep_body(n_out, x_ref, *rest):
    deg_refs = rest[:n_out]
    out_refs = rest[n_out:]
    x = x_ref[...]
    for dref, oref in zip(deg_refs, out_refs):
        _split_store(oref, x * _norm_blk(dref))


def _prep(x, degs):
    n_out = len(degs)
    outs = pl.pallas_call(
        functools.partial(_prep_body, n_out),
        grid=(NB,),
        in_specs=[_SPEC_X] + [_SPEC_DEG] * n_out,
        out_specs=[_SPEC_OUT2] * n_out,
        out_shape=[_T2] * n_out,
    )(x, *degs)
    outs = outs if isinstance(outs, (list, tuple)) else [outs]
    return [o.reshape(2 * NPAD, DH) for o in outs]


def _dense_body(nrel, nscale, relu, *refs):
    i = 0
    acc = None
    for _ in range(nrel):
        mref, dref, wref, bref = refs[i:i + 4]
        i += 4
        part = _path(mref, dref, wref, bref)
        acc = part if acc is None else acc + part
    if relu:
        acc = jnp.maximum(acc, 0.0)
    if nscale == 0:
        refs[i][...] = acc
    else:
        sdegs = refs[i:i + nscale]
        outs = refs[i + nscale:]
        for dref, oref in zip(sdegs, outs):
            _split_store(oref, acc * _norm_blk(dref))


def _dense(ms, degs_in, Ws, bs, scale_degs):
    nrel = len(ms)
    final = scale_degs is None
    nscale = 0 if final else len(scale_degs)
    in_specs = []
    args = []
    for m, dgr, w, b in zip(ms, degs_in, Ws, bs):
        in_specs += [_SPEC_M, _SPEC_DEG, _SPEC_W, _SPEC_B]
        args += [m.reshape(2, NPAD, DH), dgr, w, b.reshape(1, D)]
    if final:
        out_specs = pl.BlockSpec((BR, D), lambda b: (b, 0))
        out_shape = jax.ShapeDtypeStruct((NPAD, D), jnp.float32)
    else:
        in_specs += [_SPEC_DEG] * nscale
        args += list(scale_degs)
        out_specs = [_SPEC_OUT2] * nscale
        out_shape = [_T2] * nscale
    res = pl.pallas_call(
        functools.partial(_dense_body, nrel, nscale, not final),
        grid=(NB,),
        in_specs=in_specs,
        out_specs=out_specs,
        out_shape=out_shape,
    )(*args)
    if final:
        return res
    res = res if isinstance(res, (list, tuple)) else [res]
    return [o.reshape(2 * NPAD, DH) for o in res]


def kernel(x_user, x_item, ei_clicks, ei_clicked_by, ei_follows,
           W1_clicks, b1_clicks, W1_clicked_by, b1_clicked_by,
           W1_follows, b1_follows,
           W2_clicks, b2_clicks, W2_clicked_by, b2_clicked_by,
           W2_follows, b2_follows):
    pad_idx = N + (jnp.arange(EPAD - E, dtype=jnp.int32) % TRASH)

    def pad_e(a):
        return jnp.concatenate([a.astype(jnp.int32), pad_idx])

    s_cl, d_cl = pad_e(ei_clicks[0]), pad_e(ei_clicks[1])
    s_cb, d_cb = pad_e(ei_clicked_by[0]), pad_e(ei_clicked_by[1])
    s_fl, d_fl = pad_e(ei_follows[0]), pad_e(ei_follows[1])

    idx_all = jnp.concatenate([s_cl, d_cl, s_cb, d_cb, s_fl, d_fl])
    degflat = _deg_kernel(idx_all.reshape(6 * _ECH, CH))

    def src2d(a):
        return jnp.concatenate([a, a + NPAD]).reshape(2 * _ECH, CH)

    def dst2d(a):
        return a.reshape(_ECH, CH)

    s_cl, s_cb, s_fl = src2d(s_cl), src2d(s_cb), src2d(s_fl)
    d_cl, d_cb, d_fl = dst2d(d_cl), dst2d(d_cb), dst2d(d_fl)

    def dg(a):
        return degflat[a * NPAD:(a + 1) * NPAD]

    t1_cl, t1_fl = _prep(x_user, [dg(0), dg(4)])
    (t1_cb,) = _prep(x_item, [dg(2)])

    m1_cl = _conv_kernel(t1_cl, s_cl, d_cl)
    m1_cb = _conv_kernel(t1_cb, s_cb, d_cb)
    m1_fl = _conv_kernel(t1_fl, s_fl, d_fl)

    (t2_cb,) = _dense([m1_cl], [dg(1)], [W1_clicks], [b1_clicks], [dg(2)])
    t2_cl, t2_fl = _dense([m1_cb, m1_fl], [dg(3), dg(5)],
                          [W1_clicked_by, W1_follows],
                          [b1_clicked_by, b1_follows], [dg(0), dg(4)])

    m2_cl = _conv_kernel(t2_cl, s_cl, d_cl)
    m2_cb = _conv_kernel(t2_cb, s_cb, d_cb)
    m2_fl = _conv_kernel(t2_fl, s_fl, d_fl)

    h_item2 = _dense([m2_cl], [dg(1)], [W2_clicks], [b2_clicks], None)[:N]
    h_user2 = _dense([m2_cb, m2_fl], [dg(3), dg(5)],
                     [W2_clicked_by, W2_follows],
                     [b2_clicked_by, b2_follows], None)[:N]
    return (h_user2, h_item2)


# speedup vs baseline: 1.4647x; 1.0056x over previous
import functools

import jax
import jax.numpy as jnp
from jax import lax
from jax.experimental import pallas as pl
from jax.experimental.pallas import tpu as pltpu
from jax.experimental.pallas import tpu_sc as plsc

N = 25000
D = 128
DH = 64
E = 200000
NT = 16
NPAD = 25088
EPAD = 204800
ROWS_PT = NPAD // NT
EDG_PT = EPAD // NT
CH = 128
NCHUNK = EDG_PT // CH
ZR = 224
TRASH = NPAD - N
BR = 1568
NB = NPAD // BR

_MESH = plsc.VectorSubcoreMesh(core_axis_name="c", subcore_axis_name="s")


_NZ = ROWS_PT // CH
_ZREM = ROWS_PT - _NZ * CH
_ECH = EPAD // CH
_LAG = 8
_NH = NCHUNK // 2


def _deg_body(idx2d, out, onesv, zbuf, ibuf, acc0, acc1, acc2, ssem):
    c = lax.axis_index("c")
    s = lax.axis_index("s")
    accs = [acc0, acc1, acc2]
    one16 = jnp.ones((16,), jnp.float32)
    zero16 = jnp.zeros((16,), jnp.float32)

    def fill(i, _):
        onesv[i] = one16
        zbuf[i] = zero16
        return 0

    lax.fori_loop(0, CH, fill, 0)

    row0 = s * ROWS_PT
    for a in range(3):
        arr = c * 3 + a
        acc = accs[a]
        for k in range(_NZ):
            pltpu.async_copy(zbuf, acc.at[pl.ds(row0 + k * CH, CH)], ssem)
        pltpu.async_copy(zbuf.at[pl.ds(0, _ZREM)],
                         acc.at[pl.ds(row0 + _NZ * CH, _ZREM)], ssem)
        pltpu.sync_copy(idx2d.at[pl.ds(arr * _ECH + s * NCHUNK, NCHUNK)],
                        ibuf)
        for k in range(_NZ):
            pltpu.make_async_copy(zbuf, acc.at[pl.ds(0, CH)], ssem).wait()
        pltpu.make_async_copy(zbuf.at[pl.ds(0, _ZREM)],
                              acc.at[pl.ds(0, _ZREM)], ssem).wait()
        plsc.subcore_barrier()

        def chunk(i, _):
            pltpu.async_copy(onesv, acc.at[ibuf.at[i]], ssem, add=True)

            @pl.when(i >= _LAG)
            def _():
                pltpu.make_async_copy(onesv, acc.at[pl.ds(0, CH)],
                                      ssem).wait()

            return 0

        lax.fori_loop(0, NCHUNK, chunk, 0)
        for _ in range(_LAG):
            pltpu.make_async_copy(onesv, acc.at[pl.ds(0, CH)], ssem).wait()
        plsc.subcore_barrier()
        pltpu.sync_copy(acc.at[pl.ds(row0, ROWS_PT)],
                        out.at[pl.ds(arr * NPAD + row0, ROWS_PT)])


_deg_kernel = pl.kernel(
    _deg_body,
    out_type=jax.ShapeDtypeStruct((6 * NPAD, 16), jnp.float32),
    mesh=_MESH,
    scratch_types=[
        pltpu.VMEM((CH, 16), jnp.float32),
        pltpu.VMEM((CH, 16), jnp.float32),
        pltpu.VMEM((NCHUNK, CH), jnp.int32),
        pltpu.VMEM_SHARED((NPAD, 16), jnp.float32),
        pltpu.VMEM_SHARED((NPAD, 16), jnp.float32),
        pltpu.VMEM_SHARED((NPAD, 16), jnp.float32),
        pltpu.SemaphoreType.DMA,
    ],
    compiler_params=pltpu.CompilerParams(use_tc_tiling_on_sc=False),
)


def _conv_body(table, so2d, dst2d, out, sbuf, dbuf, st0, st1, acc,
               gsem, ssem):
    c = lax.axis_index("c")
    s = lax.axis_index("s")
    zero16 = jnp.zeros((16,), jnp.float32)
    row0 = s * ROWS_PT

    def wait_g():
        pltpu.make_async_copy(table.at[pl.ds(0, CH)], st0, gsem).wait()

    def wait_s():
        pltpu.make_async_copy(st0, acc.at[pl.ds(0, CH)], ssem).wait()

    if True:
        def zfill(i, _):
            for k in range(DH // 16):
                st0[i, pl.ds(k * 16, 16)] = zero16
            return 0

        lax.fori_loop(0, CH, zfill, 0)

        for k in range(_NZ):
            pltpu.async_copy(st0, acc.at[pl.ds(row0 + k * CH, CH)], ssem)
        pltpu.async_copy(st0.at[pl.ds(0, _ZREM)],
                         acc.at[pl.ds(row0 + _NZ * CH, _ZREM)], ssem)
        for k in range(_NZ):
            pltpu.make_async_copy(st0, acc.at[pl.ds(0, CH)], ssem).wait()
        pltpu.make_async_copy(st0.at[pl.ds(0, _ZREM)],
                              acc.at[pl.ds(0, _ZREM)], ssem).wait()
        plsc.subcore_barrier()

        def gat(i, buf):
            pltpu.async_copy(table.at[sbuf.at[i]], buf, gsem)

        def sca(i, buf):
            pltpu.async_copy(buf, acc.at[dbuf.at[i]], ssem, add=True)

        for h in range(2):
            pltpu.sync_copy(
                so2d.at[pl.ds((c * NT + s) * NCHUNK + h * _NH, _NH)], sbuf)
            pltpu.sync_copy(dst2d.at[pl.ds(s * NCHUNK + h * _NH, _NH)],
                            dbuf)
            gat(0, st0)

            def pair(k, _):
                i0 = 2 * k

                @pl.when(k > 0)
                def _():
                    wait_s()
                gat(i0 + 1, st1)
                wait_g()
                sca(i0, st0)
                wait_s()
                @pl.when(i0 + 2 < _NH)
                def _():
                    gat(i0 + 2, st0)
                wait_g()
                sca(i0 + 1, st1)
                return 0

            lax.fori_loop(0, _NH // 2, pair, 0)
            wait_s()
        plsc.subcore_barrier()
        pltpu.sync_copy(acc.at[pl.ds(row0, ROWS_PT)],
                        out.at[pl.ds(c * NPAD + row0, ROWS_PT)])


_conv_kernel = pl.kernel(
    _conv_body,
    out_type=jax.ShapeDtypeStruct((2 * NPAD, DH), jnp.float32),
    mesh=_MESH,
    scratch_types=[
        pltpu.VMEM((_NH, CH), jnp.int32),
        pltpu.VMEM((_NH, CH), jnp.int32),
        pltpu.VMEM((CH, DH), jnp.float32),
        pltpu.VMEM((CH, DH), jnp.float32),
        pltpu.VMEM_SHARED((NPAD, DH), jnp.float32),
        pltpu.SemaphoreType.DMA,
        pltpu.SemaphoreType.DMA,
    ],
    compiler_params=pltpu.CompilerParams(use_tc_tiling_on_sc=False),
)


def _norm_blk(deg_ref):
    return lax.rsqrt(jnp.maximum(deg_ref[:, 0:1], 1.0))


def _split_store(oref, val):
    oref[0] = val[:, 0:DH]
    oref[1] = val[:, DH:D]


_SPEC_M = pl.BlockSpec((2, BR, DH), lambda b: (0, b, 0))
_SPEC_DEG = pl.BlockSpec((BR, 16), lambda b: (b, 0))
_SPEC_W = pl.BlockSpec((D, D), lambda b: (0, 0))
_SPEC_B = pl.BlockSpec((1, D), lambda b: (0, 0))
_SPEC_X = pl.BlockSpec((BR, D), lambda b: (b, 0))
_SPEC_OUT2 = pl.BlockSpec((2, BR, DH), lambda b: (0, b, 0))
_T2 = jax.ShapeDtypeStruct((2, NPAD, DH), jnp.float32)


def _path(mref, dref, wref, bref):
    norm = _norm_blk(dref)
    return (jnp.dot(mref[0] * norm, wref[0:DH, :],
                    preferred_element_type=jnp.float32)
            + jnp.dot(mref[1] * norm, wref[DH:D, :],
                      preferred_element_type=jnp.float32)
            + bref[...])


def _prep_body(n_out, x_ref, *rest):
    deg_refs = rest[:n_out]
    out_refs = rest[n_out:]
    x = x_ref[...]
    for dref, oref in zip(deg_refs, out_refs):
        _split_store(oref, x * _norm_blk(dref))


def _prep(x, degs):
    n_out = len(degs)
    outs = pl.pallas_call(
        functools.partial(_prep_body, n_out),
        grid=(NB,),
        in_specs=[_SPEC_X] + [_SPEC_DEG] * n_out,
        out_specs=[_SPEC_OUT2] * n_out,
        out_shape=[_T2] * n_out,
    )(x, *degs)
    outs = outs if isinstance(outs, (list, tuple)) else [outs]
    return [o.reshape(2 * NPAD, DH) for o in outs]


def _dense_body(nrel, nscale, relu, *refs):
    i = 0
    acc = None
    for _ in range(nrel):
        mref, dref, wref, bref = refs[i:i + 4]
        i += 4
        part = _path(mref, dref, wref, bref)
        acc = part if acc is None else acc + part
    if relu:
        acc = jnp.maximum(acc, 0.0)
    if nscale == 0:
        refs[i][...] = acc
    else:
        sdegs = refs[i:i + nscale]
        outs = refs[i + nscale:]
        for dref, oref in zip(sdegs, outs):
            _split_store(oref, acc * _norm_blk(dref))


def _dense(ms, degs_in, Ws, bs, scale_degs):
    nrel = len(ms)
    final = scale_degs is None
    nscale = 0 if final else len(scale_degs)
    in_specs = []
    args = []
    for m, dgr, w, b in zip(ms, degs_in, Ws, bs):
        in_specs += [_SPEC_M, _SPEC_DEG, _SPEC_W, _SPEC_B]
        args += [m.reshape(2, NPAD, DH), dgr, w, b.reshape(1, D)]
    if final:
        out_specs = pl.BlockSpec((BR, D), lambda b: (b, 0))
        out_shape = jax.ShapeDtypeStruct((NPAD, D), jnp.float32)
    else:
        in_specs += [_SPEC_DEG] * nscale
        args += list(scale_degs)
        out_specs = [_SPEC_OUT2] * nscale
        out_shape = [_T2] * nscale
    res = pl.pallas_call(
        functools.partial(_dense_body, nrel, nscale, not final),
        grid=(NB,),
        in_specs=in_specs,
        out_specs=out_specs,
        out_shape=out_shape,
    )(*args)
    if final:
        return res
    res = res if isinstance(res, (list, tuple)) else [res]
    return [o.reshape(2 * NPAD, DH) for o in res]


def kernel(x_user, x_item, ei_clicks, ei_clicked_by, ei_follows,
           W1_clicks, b1_clicks, W1_clicked_by, b1_clicked_by,
           W1_follows, b1_follows,
           W2_clicks, b2_clicks, W2_clicked_by, b2_clicked_by,
           W2_follows, b2_follows):
    pad_idx = N + (jnp.arange(EPAD - E, dtype=jnp.int32) % TRASH)

    def pad_e(a):
        return jnp.concatenate([a.astype(jnp.int32), pad_idx])

    s_cl, d_cl = pad_e(ei_clicks[0]), pad_e(ei_clicks[1])
    s_cb, d_cb = pad_e(ei_clicked_by[0]), pad_e(ei_clicked_by[1])
    s_fl, d_fl = pad_e(ei_follows[0]), pad_e(ei_follows[1])

    idx_all = jnp.concatenate([s_cl, d_cl, s_cb, d_cb, s_fl, d_fl])
    degflat = _deg_kernel(idx_all.reshape(6 * _ECH, CH))

    def src2d(a):
        return jnp.concatenate([a, a + NPAD]).reshape(2 * _ECH, CH)

    def dst2d(a):
        return a.reshape(_ECH, CH)

    s_cl, s_cb, s_fl = src2d(s_cl), src2d(s_cb), src2d(s_fl)
    d_cl, d_cb, d_fl = dst2d(d_cl), dst2d(d_cb), dst2d(d_fl)

    def dg(a):
        return degflat[a * NPAD:(a + 1) * NPAD]

    t1_cl, t1_fl = _prep(x_user, [dg(0), dg(4)])
    (t1_cb,) = _prep(x_item, [dg(2)])

    m1_cl = _conv_kernel(t1_cl, s_cl, d_cl)
    m1_cb = _conv_kernel(t1_cb, s_cb, d_cb)
    m1_fl = _conv_kernel(t1_fl, s_fl, d_fl)

    (t2_cb,) = _dense([m1_cl], [dg(1)], [W1_clicks], [b1_clicks], [dg(2)])
    t2_cl, t2_fl = _dense([m1_cb, m1_fl], [dg(3), dg(5)],
                          [W1_clicked_by, W1_follows],
                          [b1_clicked_by, b1_follows], [dg(0), dg(4)])

    m2_cl = _conv_kernel(t2_cl, s_cl, d_cl)
    m2_cb = _conv_kernel(t2_cb, s_cb, d_cb)
    m2_fl = _conv_kernel(t2_fl, s_fl, d_fl)

    h_item2 = _dense([m2_cl], [dg(1)], [W2_clicks], [b2_clicks], None)[:N]
    h_user2 = _dense([m2_cb, m2_fl], [dg(3), dg(5)],
                     [W2_clicked_by, W2_follows],
                     [b2_clicked_by, b2_follows], None)[:N]
    return (h_user2, h_item2)
